# baseline (speedup 1.0000x reference)
import functools

import jax
import jax.numpy as jnp
from jax import lax
from jax.experimental import pallas as pl
from jax.experimental.pallas import tpu as pltpu
from jax.experimental.pallas import tpu_sc as plsc

_NC = 2
_NS = 16
_NW = _NC * _NS

_KI = 8
_CHUNK = _KI * 128


def _make_fused(num_rows: int, embed_dim: int, feat: int):
    assert num_rows % (_NW * _CHUNK) == 0
    out_dim = embed_dim + feat - 1
    rows_per_w = num_rows // _NW
    steps = rows_per_w // _CHUNK

    mesh = plsc.VectorSubcoreMesh(core_axis_name="c", subcore_axis_name="s")

    @functools.partial(
        pl.kernel,
        mesh=mesh,
        out_type=jax.ShapeDtypeStruct((num_rows, out_dim), jnp.float32),
        scratch_types=[
            pltpu.VMEM((_KI, 128), jnp.int32),
            pltpu.VMEM((_CHUNK, feat), jnp.float32),
            pltpu.VMEM((_CHUNK, feat - 1), jnp.float32),
            pltpu.VMEM((_CHUNK, embed_dim), jnp.float32),
            pltpu.SemaphoreType.DMA,
        ],
        compiler_params=pltpu.CompilerParams(
            use_tc_tiling_on_sc=False, needs_layout_passes=False
        ),
    )
    def fused_kernel(x_hbm, table_hbm, out_hbm, idx_v, x_v, f_v, emb_v, sem):
        wid = lax.axis_index("s") * _NC + lax.axis_index("c")
        row_base = wid * rows_per_w
        lane = lax.iota(jnp.int32, 16)
        cols = [jnp.full((16,), q, jnp.int32) for q in range(feat)]

        def step(i, carry):
            r0 = row_base + i * _CHUNK
            pltpu.sync_copy(x_hbm.at[pl.ds(r0, _CHUNK)], x_v)

            def extract(g, c):
                rows = g * 16 + lane
                ids = plsc.load_gather(x_v, [rows, cols[0]])
                idx_v[g // 8, pl.ds((g % 8) * 16, 16)] = ids.astype(jnp.int32)
                for q in range(1, feat):
                    v = plsc.load_gather(x_v, [rows, cols[q]])
                    plsc.store_scatter(f_v, [rows, cols[q - 1]], v)
                return c

            lax.fori_loop(0, _CHUNK // 16, extract, 0, unroll=4)

            copies = [
                pltpu.async_copy(
                    table_hbm.at[idx_v.at[j]],
                    emb_v.at[pl.ds(j * 128, 128)],
                    sem,
                )
                for j in range(_KI)
            ]
            for c in copies:
                c.wait()

            pltpu.sync_copy(
                emb_v, out_hbm.at[pl.ds(r0, _CHUNK), pl.ds(0, embed_dim)]
            )
            pltpu.sync_copy(
                f_v, out_hbm.at[pl.ds(r0, _CHUNK), pl.ds(embed_dim, feat - 1)]
            )
            return carry

        lax.fori_loop(0, steps, step, 0)

    return fused_kernel


def kernel(x, embed_weight):
    batch, seq, feat = x.shape
    num_rows = batch * seq
    embed_dim = embed_weight.shape[1]

    x2 = x.reshape(num_rows, feat)
    out = _make_fused(num_rows, embed_dim, feat)(x2, embed_weight)
    return out.reshape(batch, seq, embed_dim + feat - 1)
